# Initial kernel scaffold; baseline (speedup 1.0000x reference)
import jax
import jax.numpy as jnp
from jax.experimental import pallas as pl


def kernel(atomic_numbers, table):
    raise NotImplementedError("write your pallas kernel here")


# speedup vs baseline: 1.1750x; 1.1750x over previous
import functools

import jax
import jax.numpy as jnp
from jax import lax
from jax.experimental import pallas as pl
from jax.experimental.pallas import tpu as pltpu
from jax.experimental.pallas import tpu_sc as plsc

B = 100000
D = 256
C = 256
NC = 2
NS = 16
NW = NC * NS
NUM_CHUNKS = -(-B // C)
TAIL = B - (NUM_CHUNKS - 1) * C


@functools.partial(
    pl.kernel,
    mesh=plsc.VectorSubcoreMesh(core_axis_name="c", subcore_axis_name="s"),
    out_type=jax.ShapeDtypeStruct((B, D), jnp.float32),
    scratch_types=[
        pltpu.VMEM((C,), jnp.int32),
        pltpu.VMEM((C, D), jnp.float32),
        pltpu.SemaphoreType.DMA,
    ],
)
def _gather_kernel(idx_hbm, table_hbm, out_hbm, idx_v, rows_v, sem):
    wid = lax.axis_index("s") * NC + lax.axis_index("c")

    @pl.loop(wid, NUM_CHUNKS, step=NW)
    def _chunk(chunk):
        base = chunk * C

        @pl.when(chunk < NUM_CHUNKS - 1)
        def _full():
            pltpu.sync_copy(idx_hbm.at[pl.ds(base, C)], idx_v)
            pltpu.async_copy(table_hbm.at[idx_v], rows_v, sem).wait()
            pltpu.sync_copy(rows_v, out_hbm.at[pl.ds(base, C)])

        @pl.when(chunk == NUM_CHUNKS - 1)
        def _tail():
            pltpu.sync_copy(idx_hbm.at[pl.ds(base, TAIL)],
                            idx_v.at[pl.ds(0, TAIL)])
            pltpu.async_copy(table_hbm.at[idx_v.at[pl.ds(0, TAIL)]],
                             rows_v.at[pl.ds(0, TAIL)], sem).wait()
            pltpu.sync_copy(rows_v.at[pl.ds(0, TAIL)],
                            out_hbm.at[pl.ds(base, TAIL)])


def kernel(atomic_numbers, table):
    idx = atomic_numbers.astype(jnp.int32)
    return _gather_kernel(idx, table)
